# Initial kernel scaffold; baseline (speedup 1.0000x reference)
import jax
import jax.numpy as jnp
from jax.experimental import pallas as pl


def kernel(x, edge_index, W1, att_src1, att_dst1, b1, W2, att_src2, att_dst2, b2):
    raise NotImplementedError("write your pallas kernel here")


# speedup vs baseline: 49.6223x; 49.6223x over previous
import functools

import jax
import jax.numpy as jnp
from jax import lax
from jax.experimental import pallas as pl
from jax.experimental.pallas import tpu as pltpu
from jax.experimental.pallas import tpu_sc as plsc

N = 10000
D_IN = 128
HEADS1 = 8
C1 = 16
D1 = HEADS1 * C1
D_OUT = 64

NP = 10240
NC, NS, LANES = 2, 16, 16
NW = NC * NS
CH = 128
E_TOT = 320000 + N
EW = 10368
EP = NW * EW
NCHUNK = EW // CH
ROWS_PER_TILE = NP // NS


def _sc_edge_pass(feat_d, heads):
    d = feat_d
    w = d + 16
    vh = d // heads // LANES
    mesh = plsc.VectorSubcoreMesh(core_axis_name="c", subcore_axis_name="s")

    @functools.partial(
        pl.kernel,
        out_type=(jax.ShapeDtypeStruct((NP, w), jnp.float32),
                  jax.ShapeDtypeStruct((NP, w), jnp.float32)),
        mesh=mesh,
        scratch_types=[
            pltpu.VMEM((CH,), jnp.int32),
            pltpu.VMEM((CH,), jnp.int32),
            pltpu.VMEM((CH, w), jnp.float32),
            pltpu.VMEM((CH, 16), jnp.float32),
            pltpu.VMEM((CH, 16), jnp.float32),
            pltpu.VMEM_SHARED((NP, w), jnp.float32),
            pltpu.SemaphoreType.DMA,
            pltpu.SemaphoreType.DMA,
        ],
        compiler_params=pltpu.CompilerParams(use_tc_tiling_on_sc=False,
                                             needs_layout_passes=False),
    )
    def edge_pass(t_hbm, aux_hbm, src_hbm, dst_hbm, out0, out1,
                  idx_s, idx_d, srow, drow, ex, acc, sem1, sem2):
        cid = lax.axis_index("c")
        sid = lax.axis_index("s")
        zero16 = jnp.zeros((LANES,), jnp.float32)

        def zero_body(e, carry):
            for v in range(w // LANES):
                srow[e, pl.ds(v * LANES, LANES)] = zero16
            ex[e, pl.ds(0, LANES)] = zero16
            return carry
        lax.fori_loop(0, CH, zero_body, 0)
        tile_base = sid * ROWS_PER_TILE
        for k in range(ROWS_PER_TILE // CH):
            pltpu.sync_copy(srow, acc.at[pl.ds(tile_base + k * CH, CH)])
        plsc.subcore_barrier()

        base_edges = (cid * NS + sid) * EW

        def chunk_body(j, carry):
            base = base_edges + j * CH
            pltpu.sync_copy(src_hbm.at[pl.ds(base, CH)], idx_s)
            pltpu.sync_copy(dst_hbm.at[pl.ds(base, CH)], idx_d)
            cp1 = pltpu.async_copy(t_hbm.at[idx_s], srow, sem1)
            cp2 = pltpu.async_copy(aux_hbm.at[idx_d], drow, sem2)
            cp1.wait()
            cp2.wait()
            for g in range(CH // LANES):
                rows = lax.iota(jnp.int32, LANES) + g * LANES
                for h in range(heads):
                    vs = plsc.load_gather(
                        srow, [rows, jnp.full((LANES,), d + h, jnp.int32)])
                    vd = plsc.load_gather(
                        drow, [rows, jnp.full((LANES,), 8 + h, jnp.int32)])
                    al = vs + vd
                    al = jnp.where(al >= 0.0, al, 0.2 * al)
                    plsc.store_scatter(
                        ex, [rows, jnp.full((LANES,), h, jnp.int32)],
                        jnp.exp(al))

            def msg_body(e, carry):
                exv = ex[e, pl.ds(0, LANES)]
                srow[e, pl.ds(d, LANES)] = exv
                for h in range(heads):
                    s = exv[h]
                    for v in range(vh):
                        col = (h * vh + v) * LANES
                        srow[e, pl.ds(col, LANES)] = (
                            srow[e, pl.ds(col, LANES)] * s)
                return carry
            lax.fori_loop(0, CH, msg_body, 0)
            pltpu.sync_copy(srow, acc.at[idx_d], add=True)
            return carry
        lax.fori_loop(0, NCHUNK, chunk_body, 0)

        plsc.subcore_barrier()
        my_rows = pl.ds(tile_base, ROWS_PER_TILE)

        @pl.when(cid == 0)
        def _():
            pltpu.sync_copy(acc.at[my_rows], out0.at[my_rows])

        @pl.when(cid == 1)
        def _():
            pltpu.sync_copy(acc.at[my_rows], out1.at[my_rows])

    return edge_pass


_edge_pass1 = _sc_edge_pass(D1, HEADS1)
_edge_pass2 = _sc_edge_pass(D_OUT, 1)


def _mm1_body(x_ref, w_ref, as_ref, ad_ref, xw_ref, asrc_ref, adst_ref):
    xw = jnp.dot(x_ref[...], w_ref[...], preferred_element_type=jnp.float32)
    xw_ref[...] = xw
    asrc_ref[...] = jnp.dot(xw, as_ref[...], preferred_element_type=jnp.float32)
    adst_ref[...] = jnp.dot(xw, ad_ref[...], preferred_element_type=jnp.float32)


def _mid_body(a0_ref, a1_ref, r1_ref, b1_ref, w2_ref, a2_ref,
              h2_ref, aux2_ref):
    acc = a0_ref[...] + a1_ref[...]
    num = acc[:, :D1]
    den = jnp.dot(acc[:, D1:], r1_ref[...], preferred_element_type=jnp.float32)
    h1 = num / (den + 1e-16) + b1_ref[...]
    h1 = jnp.where(h1 > 0.0, h1, jnp.exp(h1) - 1.0)
    h2 = jnp.dot(h1, w2_ref[...], preferred_element_type=jnp.float32)
    h2_ref[...] = h2
    aux2_ref[...] = jnp.dot(h2, a2_ref[...], preferred_element_type=jnp.float32)


def _final_body(a0_ref, a1_ref, r2_ref, b2_ref, h_ref, lp_ref):
    acc = a0_ref[...] + a1_ref[...]
    num = acc[:, :D_OUT]
    den = jnp.dot(acc[:, D_OUT:], r2_ref[...],
                  preferred_element_type=jnp.float32)
    h = num / (den + 1e-16) + b2_ref[...]
    h_ref[...] = h
    m = jnp.max(h, axis=1, keepdims=True)
    lse = jnp.log(jnp.sum(jnp.exp(h - m), axis=1, keepdims=True)) + m
    lp_ref[...] = h - lse


def kernel(x, edge_index, W1, att_src1, att_dst1, b1, W2, att_src2,
           att_dst2, b2):
    f32 = jnp.float32
    loop = jnp.arange(N, dtype=jnp.int32)
    pad_e = EP - E_TOT
    src = jnp.concatenate([edge_index[0], loop,
                           jnp.zeros((pad_e,), jnp.int32)])
    dst = jnp.concatenate([edge_index[1], loop,
                           jnp.full((pad_e,), N, jnp.int32)])
    x_pad = jnp.concatenate([x, jnp.zeros((NP - N, D_IN), f32)], axis=0)
    eye1 = jnp.eye(HEADS1, dtype=f32)
    a_s1 = (att_src1[:, :, None] * eye1[:, None, :]).reshape(D1, HEADS1)
    a_d1 = (att_dst1[:, :, None] * eye1[:, None, :]).reshape(D1, HEADS1)
    r1 = jnp.concatenate(
        [jnp.kron(jnp.eye(HEADS1, dtype=f32), jnp.ones((1, C1), f32)),
         jnp.zeros((16 - HEADS1, D1), f32)], axis=0)
    r2 = jnp.concatenate([jnp.ones((1, D_OUT), f32),
                          jnp.zeros((15, D_OUT), f32)], axis=0)
    a2 = jnp.concatenate(
        [att_src2.T, jnp.zeros((D_OUT, 7), f32),
         att_dst2.T, jnp.zeros((D_OUT, 7), f32)], axis=1)
    b1r = b1.reshape(1, D1)
    b2r = b2.reshape(1, D_OUT)

    blk = 512
    g1 = NP // blk
    xw, asrc, adst = pl.pallas_call(
        _mm1_body,
        grid=(g1,),
        in_specs=[
            pl.BlockSpec((blk, D_IN), lambda i: (i, 0)),
            pl.BlockSpec((D_IN, D1), lambda i: (0, 0)),
            pl.BlockSpec((D1, HEADS1), lambda i: (0, 0)),
            pl.BlockSpec((D1, HEADS1), lambda i: (0, 0)),
        ],
        out_specs=[
            pl.BlockSpec((blk, D1), lambda i: (i, 0)),
            pl.BlockSpec((blk, HEADS1), lambda i: (i, 0)),
            pl.BlockSpec((blk, HEADS1), lambda i: (i, 0)),
        ],
        out_shape=[
            jax.ShapeDtypeStruct((NP, D1), f32),
            jax.ShapeDtypeStruct((NP, HEADS1), f32),
            jax.ShapeDtypeStruct((NP, HEADS1), f32),
        ],
    )(x_pad, W1, a_s1, a_d1)

    aux1 = jnp.concatenate([asrc, adst], axis=1)
    t1 = jnp.concatenate([xw, aux1], axis=1)

    acc1a, acc1b = _edge_pass1(t1, aux1, src, dst)

    w1acc = D1 + 16
    h2, aux2 = pl.pallas_call(
        _mid_body,
        grid=(g1,),
        in_specs=[
            pl.BlockSpec((blk, w1acc), lambda i: (i, 0)),
            pl.BlockSpec((blk, w1acc), lambda i: (i, 0)),
            pl.BlockSpec((16, D1), lambda i: (0, 0)),
            pl.BlockSpec((1, D1), lambda i: (0, 0)),
            pl.BlockSpec((D1, D_OUT), lambda i: (0, 0)),
            pl.BlockSpec((D_OUT, 16), lambda i: (0, 0)),
        ],
        out_specs=[
            pl.BlockSpec((blk, D_OUT), lambda i: (i, 0)),
            pl.BlockSpec((blk, 16), lambda i: (i, 0)),
        ],
        out_shape=[
            jax.ShapeDtypeStruct((NP, D_OUT), f32),
            jax.ShapeDtypeStruct((NP, 16), f32),
        ],
    )(acc1a, acc1b, r1, b1r, W2, a2)

    t2 = jnp.concatenate([h2, aux2], axis=1)

    acc2a, acc2b = _edge_pass2(t2, aux2, src, dst)

    blk_c = 1000
    g2 = N // blk_c
    w2acc = D_OUT + 16
    h, logp = pl.pallas_call(
        _final_body,
        grid=(g2,),
        in_specs=[
            pl.BlockSpec((blk_c, w2acc), lambda i: (i, 0)),
            pl.BlockSpec((blk_c, w2acc), lambda i: (i, 0)),
            pl.BlockSpec((16, D_OUT), lambda i: (0, 0)),
            pl.BlockSpec((1, D_OUT), lambda i: (0, 0)),
        ],
        out_specs=[
            pl.BlockSpec((blk_c, D_OUT), lambda i: (i, 0)),
            pl.BlockSpec((blk_c, D_OUT), lambda i: (i, 0)),
        ],
        out_shape=[
            jax.ShapeDtypeStruct((N, D_OUT), f32),
            jax.ShapeDtypeStruct((N, D_OUT), f32),
        ],
    )(acc2a, acc2b, r2, b2r)

    return (h, logp)


# speedup vs baseline: 59.4544x; 1.1981x over previous
import functools

import jax
import jax.numpy as jnp
from jax import lax
from jax.experimental import pallas as pl
from jax.experimental.pallas import tpu as pltpu
from jax.experimental.pallas import tpu_sc as plsc

N = 10000
D_IN = 128
HEADS1 = 8
C1 = 16
D1 = HEADS1 * C1
D_OUT = 64

NP = 10240
NC, NS, LANES = 2, 16, 16
NW = NC * NS
CH = 96
E_TOT = 320000 + N
EW = 10368
EP = NW * EW
NCHUNK = EW // CH
ROWS_PER_TILE = NP // NS


def _sc_edge_pass(feat_d, heads):
    d = feat_d
    w = d + 16
    vh = d // heads // LANES
    mesh = plsc.VectorSubcoreMesh(core_axis_name="c", subcore_axis_name="s")

    @functools.partial(
        pl.kernel,
        out_type=(jax.ShapeDtypeStruct((NP, w), jnp.float32),
                  jax.ShapeDtypeStruct((NP, w), jnp.float32)),
        mesh=mesh,
        scratch_types=[
            [pltpu.VMEM((CH,), jnp.int32)] * 2,
            [pltpu.VMEM((CH,), jnp.int32)] * 2,
            [pltpu.VMEM((CH,), jnp.int32)] * 2,
            [pltpu.VMEM((CH, w), jnp.float32)] * 2,
            [pltpu.VMEM((CH, 16), jnp.float32)] * 2,
            pltpu.VMEM((CH, 16), jnp.float32),
            pltpu.VMEM_SHARED((NP, w), jnp.float32),
            [pltpu.SemaphoreType.DMA] * 2,
            [pltpu.SemaphoreType.DMA] * 2,
            [pltpu.SemaphoreType.DMA] * 2,
            [pltpu.SemaphoreType.DMA] * 2,
            [pltpu.SemaphoreType.DMA] * 2,
            [pltpu.SemaphoreType.DMA] * 2,
        ],
        compiler_params=pltpu.CompilerParams(use_tc_tiling_on_sc=False,
                                             needs_layout_passes=False),
    )
    def edge_pass(t_hbm, aux_hbm, src_hbm, dst_hbm, out0, out1,
                  idx_s, idx_da, idx_db, srow, drow, ex, acc,
                  sgt, sga, sis, sia, sib, ss):
        cid = lax.axis_index("c")
        sid = lax.axis_index("s")
        zero16 = jnp.zeros((LANES,), jnp.float32)

        def zero_body(e, carry):
            for v in range(w // LANES):
                srow[0][e, pl.ds(v * LANES, LANES)] = zero16
            ex[e, pl.ds(0, LANES)] = zero16
            return carry
        lax.fori_loop(0, CH, zero_body, 0)
        tile_base = sid * ROWS_PER_TILE
        nfull = ROWS_PER_TILE // CH
        for k in range(nfull):
            pltpu.sync_copy(srow[0], acc.at[pl.ds(tile_base + k * CH, CH)])
        rem = ROWS_PER_TILE - nfull * CH
        if rem:
            pltpu.sync_copy(
                srow[0].at[pl.ds(0, rem)],
                acc.at[pl.ds(tile_base + nfull * CH, rem)])
        plsc.subcore_barrier()

        base_edges = (cid * NS + sid) * EW

        def load_idx_s(j, b):
            pltpu.async_copy(
                src_hbm.at[pl.ds(base_edges + j * CH, CH)], idx_s[b], sis[b])

        def load_idx_da(j, b):
            pltpu.async_copy(
                dst_hbm.at[pl.ds(base_edges + j * CH, CH)], idx_da[b], sia[b])

        def load_idx_db(j, b):
            pltpu.async_copy(
                dst_hbm.at[pl.ds(base_edges + j * CH, CH)], idx_db[b], sib[b])

        def issue_gathers(b):
            pltpu.async_copy(t_hbm.at[idx_s[b]], srow[b], sgt[b])
            pltpu.async_copy(aux_hbm.at[idx_da[b]], drow[b], sga[b])

        def wait_gathers(b):
            pltpu.make_async_copy(t_hbm.at[idx_s[b]], srow[b], sgt[b]).wait()
            pltpu.make_async_copy(aux_hbm.at[idx_da[b]], drow[b],
                                  sga[b]).wait()

        def wait_idx_s(b):
            pltpu.make_async_copy(
                src_hbm.at[pl.ds(0, CH)], idx_s[b], sis[b]).wait()

        def wait_idx_da(b):
            pltpu.make_async_copy(
                dst_hbm.at[pl.ds(0, CH)], idx_da[b], sia[b]).wait()

        def wait_idx_db(b):
            pltpu.make_async_copy(
                dst_hbm.at[pl.ds(0, CH)], idx_db[b], sib[b]).wait()

        def wait_scatter(b):
            pltpu.make_async_copy(srow[b], acc.at[idx_db[b]], ss[b]).wait()

        def compute(b):
            sr, dr = srow[b], drow[b]
            for g in range(CH // LANES):
                rows = lax.iota(jnp.int32, LANES) + g * LANES
                for h in range(heads):
                    vs = plsc.load_gather(
                        sr, [rows, jnp.full((LANES,), d + h, jnp.int32)])
                    vd = plsc.load_gather(
                        dr, [rows, jnp.full((LANES,), 8 + h, jnp.int32)])
                    al = vs + vd
                    al = jnp.where(al >= 0.0, al, 0.2 * al)
                    plsc.store_scatter(
                        ex, [rows, jnp.full((LANES,), h, jnp.int32)],
                        jnp.exp(al))

            def msg_body(e, carry):
                exv = ex[e, pl.ds(0, LANES)]
                sr[e, pl.ds(d, LANES)] = exv
                for h in range(heads):
                    s = exv[h]
                    for v in range(vh):
                        col = (h * vh + v) * LANES
                        sr[e, pl.ds(col, LANES)] = (
                            sr[e, pl.ds(col, LANES)] * s)
                return carry
            lax.fori_loop(0, CH, msg_body, 0)

        def step(j, b, nb):
            wait_gathers(b)

            @pl.when(j + 2 < NCHUNK)
            def _():
                load_idx_s(j + 2, b)
                load_idx_da(j + 2, b)
            compute(b)

            @pl.when(j >= 1)
            def _():
                wait_idx_db(b)
            pltpu.async_copy(srow[b], acc.at[idx_db[b]], ss[b], add=True)

            @pl.when(j + 1 < NCHUNK)
            def _():
                wait_idx_s(nb)
                wait_idx_da(nb)

                @pl.when(j >= 1)
                def _():
                    wait_scatter(nb)
                load_idx_db(j + 1, nb)
                issue_gathers(nb)

        pltpu.sync_copy(src_hbm.at[pl.ds(base_edges, CH)], idx_s[0])
        pltpu.sync_copy(dst_hbm.at[pl.ds(base_edges, CH)], idx_da[0])
        pltpu.sync_copy(dst_hbm.at[pl.ds(base_edges, CH)], idx_db[0])
        issue_gathers(0)
        load_idx_s(1, 1)
        load_idx_da(1, 1)

        def pair_body(j2, carry):
            j = j2 * 2
            step(j, 0, 1)
            step(j + 1, 1, 0)
            return carry
        lax.fori_loop(0, NCHUNK // 2, pair_body, 0)
        wait_scatter(0)
        wait_scatter(1)

        plsc.subcore_barrier()
        my_rows = pl.ds(tile_base, ROWS_PER_TILE)

        @pl.when(cid == 0)
        def _():
            pltpu.sync_copy(acc.at[my_rows], out0.at[my_rows])

        @pl.when(cid == 1)
        def _():
            pltpu.sync_copy(acc.at[my_rows], out1.at[my_rows])

    return edge_pass


_edge_pass1 = _sc_edge_pass(D1, HEADS1)
_edge_pass2 = _sc_edge_pass(D_OUT, 1)


def _mm1_body(x_ref, w_ref, as_ref, ad_ref, xw_ref, asrc_ref, adst_ref):
    xw = jnp.dot(x_ref[...], w_ref[...], preferred_element_type=jnp.float32)
    xw_ref[...] = xw
    asrc_ref[...] = jnp.dot(xw, as_ref[...], preferred_element_type=jnp.float32)
    adst_ref[...] = jnp.dot(xw, ad_ref[...], preferred_element_type=jnp.float32)


def _mid_body(a0_ref, a1_ref, r1_ref, b1_ref, w2_ref, a2_ref,
              h2_ref, aux2_ref):
    acc = a0_ref[...] + a1_ref[...]
    num = acc[:, :D1]
    den = jnp.dot(acc[:, D1:], r1_ref[...], preferred_element_type=jnp.float32)
    h1 = num / (den + 1e-16) + b1_ref[...]
    h1 = jnp.where(h1 > 0.0, h1, jnp.exp(h1) - 1.0)
    h2 = jnp.dot(h1, w2_ref[...], preferred_element_type=jnp.float32)
    h2_ref[...] = h2
    aux2_ref[...] = jnp.dot(h2, a2_ref[...], preferred_element_type=jnp.float32)


def _final_body(a0_ref, a1_ref, r2_ref, b2_ref, h_ref, lp_ref):
    acc = a0_ref[...] + a1_ref[...]
    num = acc[:, :D_OUT]
    den = jnp.dot(acc[:, D_OUT:], r2_ref[...],
                  preferred_element_type=jnp.float32)
    h = num / (den + 1e-16) + b2_ref[...]
    h_ref[...] = h
    m = jnp.max(h, axis=1, keepdims=True)
    lse = jnp.log(jnp.sum(jnp.exp(h - m), axis=1, keepdims=True)) + m
    lp_ref[...] = h - lse


def kernel(x, edge_index, W1, att_src1, att_dst1, b1, W2, att_src2,
           att_dst2, b2):
    f32 = jnp.float32
    loop = jnp.arange(N, dtype=jnp.int32)
    pad_e = EP - E_TOT
    src = jnp.concatenate([edge_index[0], loop,
                           jnp.zeros((pad_e,), jnp.int32)])
    dst = jnp.concatenate([edge_index[1], loop,
                           jnp.full((pad_e,), N, jnp.int32)])
    x_pad = jnp.concatenate([x, jnp.zeros((NP - N, D_IN), f32)], axis=0)
    eye1 = jnp.eye(HEADS1, dtype=f32)
    a_s1 = (att_src1[:, :, None] * eye1[:, None, :]).reshape(D1, HEADS1)
    a_d1 = (att_dst1[:, :, None] * eye1[:, None, :]).reshape(D1, HEADS1)
    r1 = jnp.concatenate(
        [jnp.kron(jnp.eye(HEADS1, dtype=f32), jnp.ones((1, C1), f32)),
         jnp.zeros((16 - HEADS1, D1), f32)], axis=0)
    r2 = jnp.concatenate([jnp.ones((1, D_OUT), f32),
                          jnp.zeros((15, D_OUT), f32)], axis=0)
    a2 = jnp.concatenate(
        [att_src2.T, jnp.zeros((D_OUT, 7), f32),
         att_dst2.T, jnp.zeros((D_OUT, 7), f32)], axis=1)
    b1r = b1.reshape(1, D1)
    b2r = b2.reshape(1, D_OUT)

    blk = 512
    g1 = NP // blk
    xw, asrc, adst = pl.pallas_call(
        _mm1_body,
        grid=(g1,),
        in_specs=[
            pl.BlockSpec((blk, D_IN), lambda i: (i, 0)),
            pl.BlockSpec((D_IN, D1), lambda i: (0, 0)),
            pl.BlockSpec((D1, HEADS1), lambda i: (0, 0)),
            pl.BlockSpec((D1, HEADS1), lambda i: (0, 0)),
        ],
        out_specs=[
            pl.BlockSpec((blk, D1), lambda i: (i, 0)),
            pl.BlockSpec((blk, HEADS1), lambda i: (i, 0)),
            pl.BlockSpec((blk, HEADS1), lambda i: (i, 0)),
        ],
        out_shape=[
            jax.ShapeDtypeStruct((NP, D1), f32),
            jax.ShapeDtypeStruct((NP, HEADS1), f32),
            jax.ShapeDtypeStruct((NP, HEADS1), f32),
        ],
    )(x_pad, W1, a_s1, a_d1)

    aux1 = jnp.concatenate([asrc, adst], axis=1)
    t1 = jnp.concatenate([xw, aux1], axis=1)

    acc1a, acc1b = _edge_pass1(t1, aux1, src, dst)

    w1acc = D1 + 16
    h2, aux2 = pl.pallas_call(
        _mid_body,
        grid=(g1,),
        in_specs=[
            pl.BlockSpec((blk, w1acc), lambda i: (i, 0)),
            pl.BlockSpec((blk, w1acc), lambda i: (i, 0)),
            pl.BlockSpec((16, D1), lambda i: (0, 0)),
            pl.BlockSpec((1, D1), lambda i: (0, 0)),
            pl.BlockSpec((D1, D_OUT), lambda i: (0, 0)),
            pl.BlockSpec((D_OUT, 16), lambda i: (0, 0)),
        ],
        out_specs=[
            pl.BlockSpec((blk, D_OUT), lambda i: (i, 0)),
            pl.BlockSpec((blk, 16), lambda i: (i, 0)),
        ],
        out_shape=[
            jax.ShapeDtypeStruct((NP, D_OUT), f32),
            jax.ShapeDtypeStruct((NP, 16), f32),
        ],
    )(acc1a, acc1b, r1, b1r, W2, a2)

    t2 = jnp.concatenate([h2, aux2], axis=1)

    acc2a, acc2b = _edge_pass2(t2, aux2, src, dst)

    blk_c = 1000
    g2 = N // blk_c
    w2acc = D_OUT + 16
    h, logp = pl.pallas_call(
        _final_body,
        grid=(g2,),
        in_specs=[
            pl.BlockSpec((blk_c, w2acc), lambda i: (i, 0)),
            pl.BlockSpec((blk_c, w2acc), lambda i: (i, 0)),
            pl.BlockSpec((16, D_OUT), lambda i: (0, 0)),
            pl.BlockSpec((1, D_OUT), lambda i: (0, 0)),
        ],
        out_specs=[
            pl.BlockSpec((blk_c, D_OUT), lambda i: (i, 0)),
            pl.BlockSpec((blk_c, D_OUT), lambda i: (i, 0)),
        ],
        out_shape=[
            jax.ShapeDtypeStruct((N, D_OUT), f32),
            jax.ShapeDtypeStruct((N, D_OUT), f32),
        ],
    )(acc2a, acc2b, r2, b2r)

    return (h, logp)


# speedup vs baseline: 78.3997x; 1.3187x over previous
import functools

import jax
import jax.numpy as jnp
from jax import lax
from jax.experimental import pallas as pl
from jax.experimental.pallas import tpu as pltpu
from jax.experimental.pallas import tpu_sc as plsc

N = 10000
D_IN = 128
HEADS1 = 8
C1 = 16
D1 = HEADS1 * C1
D_OUT = 64

NP = 10240
NC, NS, LANES = 2, 16, 16
NW = NC * NS
CH = 96
E_TOT = 320000 + N
EW = 10368
EP = NW * EW
NCHUNK = EW // CH
ROWS_PER_TILE = NP // NS


def _sc_edge_pass(feat_d, heads):
    d = feat_d
    w = d + 16
    vh = d // heads // LANES
    mesh = plsc.VectorSubcoreMesh(core_axis_name="c", subcore_axis_name="s")

    @functools.partial(
        pl.kernel,
        out_type=(jax.ShapeDtypeStruct((NP, w), jnp.float32),
                  jax.ShapeDtypeStruct((NP, w), jnp.float32)),
        mesh=mesh,
        scratch_types=[
            [pltpu.VMEM((CH,), jnp.int32)] * 2,
            [pltpu.VMEM((CH,), jnp.int32)] * 2,
            [pltpu.VMEM((CH,), jnp.int32)] * 2,
            [pltpu.VMEM((CH, w), jnp.float32)] * 2,
            [pltpu.VMEM((CH, 16), jnp.float32)] * 2,
            pltpu.VMEM((CH, 16), jnp.float32),
            pltpu.VMEM_SHARED((NP, w), jnp.float32),
            [pltpu.SemaphoreType.DMA] * 2,
            [pltpu.SemaphoreType.DMA] * 2,
            [pltpu.SemaphoreType.DMA] * 2,
            [pltpu.SemaphoreType.DMA] * 2,
            [pltpu.SemaphoreType.DMA] * 2,
            [pltpu.SemaphoreType.DMA] * 2,
        ],
        compiler_params=pltpu.CompilerParams(use_tc_tiling_on_sc=False,
                                             needs_layout_passes=False),
    )
    def edge_pass(t_hbm, aux_hbm, src_hbm, dst_hbm, out0, out1,
                  idx_s, idx_da, idx_db, srow, drow, ex, acc,
                  sgt, sga, sis, sia, sib, ss):
        cid = lax.axis_index("c")
        sid = lax.axis_index("s")
        zero16 = jnp.zeros((LANES,), jnp.float32)

        def zero_body(e, carry):
            for v in range(w // LANES):
                srow[0][e, pl.ds(v * LANES, LANES)] = zero16
            ex[e, pl.ds(0, LANES)] = zero16
            return carry
        lax.fori_loop(0, CH, zero_body, 0)
        tile_base = sid * ROWS_PER_TILE
        nfull = ROWS_PER_TILE // CH
        for k in range(nfull):
            pltpu.sync_copy(srow[0], acc.at[pl.ds(tile_base + k * CH, CH)])
        rem = ROWS_PER_TILE - nfull * CH
        if rem:
            pltpu.sync_copy(
                srow[0].at[pl.ds(0, rem)],
                acc.at[pl.ds(tile_base + nfull * CH, rem)])
        plsc.subcore_barrier()

        base_edges = (cid * NS + sid) * EW

        def load_idx_s(j, b):
            pltpu.async_copy(
                src_hbm.at[pl.ds(base_edges + j * CH, CH)], idx_s[b], sis[b])

        def load_idx_da(j, b):
            pltpu.async_copy(
                dst_hbm.at[pl.ds(base_edges + j * CH, CH)], idx_da[b], sia[b])

        def load_idx_db(j, b):
            pltpu.async_copy(
                dst_hbm.at[pl.ds(base_edges + j * CH, CH)], idx_db[b], sib[b])

        def issue_gathers(b):
            pltpu.async_copy(t_hbm.at[idx_s[b]], srow[b], sgt[b])
            pltpu.async_copy(aux_hbm.at[idx_da[b]], drow[b], sga[b])

        def wait_gathers(b):
            pltpu.make_async_copy(t_hbm.at[idx_s[b]], srow[b], sgt[b]).wait()
            pltpu.make_async_copy(aux_hbm.at[idx_da[b]], drow[b],
                                  sga[b]).wait()

        def wait_idx_s(b):
            pltpu.make_async_copy(
                src_hbm.at[pl.ds(0, CH)], idx_s[b], sis[b]).wait()

        def wait_idx_da(b):
            pltpu.make_async_copy(
                dst_hbm.at[pl.ds(0, CH)], idx_da[b], sia[b]).wait()

        def wait_idx_db(b):
            pltpu.make_async_copy(
                dst_hbm.at[pl.ds(0, CH)], idx_db[b], sib[b]).wait()

        def wait_scatter(b):
            pltpu.make_async_copy(srow[b], acc.at[idx_db[b]], ss[b]).wait()

        def compute(b):
            sr, dr = srow[b], drow[b]
            for g in range(CH // LANES):
                rows = lax.iota(jnp.int32, LANES) + g * LANES
                for h in range(heads):
                    vs = plsc.load_gather(
                        sr, [rows, jnp.full((LANES,), d + h, jnp.int32)])
                    vd = plsc.load_gather(
                        dr, [rows, jnp.full((LANES,), 8 + h, jnp.int32)])
                    al = vs + vd
                    al = jnp.where(al >= 0.0, al, 0.2 * al)
                    plsc.store_scatter(
                        ex, [rows, jnp.full((LANES,), h, jnp.int32)],
                        jnp.exp(al))

            def msg_body(e, carry):
                exv = ex[e, pl.ds(0, LANES)]
                sr[e, pl.ds(d, LANES)] = exv
                for h in range(heads):
                    s = exv[h]
                    for v in range(vh):
                        col = (h * vh + v) * LANES
                        sr[e, pl.ds(col, LANES)] = (
                            sr[e, pl.ds(col, LANES)] * s)
                return carry
            lax.fori_loop(0, CH, msg_body, 0)

        def step(j, b, nb):
            wait_gathers(b)

            @pl.when(j + 2 < NCHUNK)
            def _():
                load_idx_s(j + 2, b)
                load_idx_da(j + 2, b)

            @pl.when(j + 1 < NCHUNK)
            def _():
                wait_idx_s(nb)
                wait_idx_da(nb)

                @pl.when(j >= 1)
                def _():
                    wait_scatter(nb)
                load_idx_db(j + 1, nb)
                issue_gathers(nb)
            compute(b)

            @pl.when(j >= 1)
            def _():
                wait_idx_db(b)
            pltpu.async_copy(srow[b], acc.at[idx_db[b]], ss[b], add=True)

        pltpu.sync_copy(src_hbm.at[pl.ds(base_edges, CH)], idx_s[0])
        pltpu.sync_copy(dst_hbm.at[pl.ds(base_edges, CH)], idx_da[0])
        pltpu.sync_copy(dst_hbm.at[pl.ds(base_edges, CH)], idx_db[0])
        issue_gathers(0)
        load_idx_s(1, 1)
        load_idx_da(1, 1)

        def pair_body(j2, carry):
            j = j2 * 2
            step(j, 0, 1)
            step(j + 1, 1, 0)
            return carry
        lax.fori_loop(0, NCHUNK // 2, pair_body, 0)
        wait_scatter(0)
        wait_scatter(1)

        plsc.subcore_barrier()
        my_rows = pl.ds(tile_base, ROWS_PER_TILE)

        @pl.when(cid == 0)
        def _():
            pltpu.sync_copy(acc.at[my_rows], out0.at[my_rows])

        @pl.when(cid == 1)
        def _():
            pltpu.sync_copy(acc.at[my_rows], out1.at[my_rows])

    return edge_pass


_edge_pass1 = _sc_edge_pass(D1, HEADS1)
_edge_pass2 = _sc_edge_pass(D_OUT, 1)


def _mm1_body(x_ref, w_ref, as_ref, ad_ref, t1_ref, aux1_ref):
    xw = jnp.dot(x_ref[...], w_ref[...], preferred_element_type=jnp.float32)
    asrc = jnp.dot(xw, as_ref[...], preferred_element_type=jnp.float32)
    adst = jnp.dot(xw, ad_ref[...], preferred_element_type=jnp.float32)
    aux = jnp.concatenate([asrc, adst], axis=1)
    t1_ref[...] = jnp.concatenate([xw, aux], axis=1)
    aux1_ref[...] = aux


def _mid_body(a0_ref, a1_ref, r1_ref, b1_ref, w2_ref, a2_ref,
              t2_ref, aux2_ref):
    acc = a0_ref[...] + a1_ref[...]
    num = acc[:, :D1]
    den = jnp.dot(acc[:, D1:], r1_ref[...], preferred_element_type=jnp.float32)
    h1 = num / (den + 1e-16) + b1_ref[...]
    h1 = jnp.where(h1 > 0.0, h1, jnp.exp(h1) - 1.0)
    h2 = jnp.dot(h1, w2_ref[...], preferred_element_type=jnp.float32)
    aux2 = jnp.dot(h2, a2_ref[...], preferred_element_type=jnp.float32)
    t2_ref[...] = jnp.concatenate([h2, aux2], axis=1)
    aux2_ref[...] = aux2


def _final_body(a0_ref, a1_ref, r2_ref, b2_ref, h_ref, lp_ref):
    acc = a0_ref[...] + a1_ref[...]
    num = acc[:, :D_OUT]
    den = jnp.dot(acc[:, D_OUT:], r2_ref[...],
                  preferred_element_type=jnp.float32)
    h = num / (den + 1e-16) + b2_ref[...]
    h_ref[...] = h
    m = jnp.max(h, axis=1, keepdims=True)
    lse = jnp.log(jnp.sum(jnp.exp(h - m), axis=1, keepdims=True)) + m
    lp_ref[...] = h - lse


def kernel(x, edge_index, W1, att_src1, att_dst1, b1, W2, att_src2,
           att_dst2, b2):
    f32 = jnp.float32
    loop = jnp.arange(N, dtype=jnp.int32)
    pad_e = EP - E_TOT
    src = jnp.concatenate([edge_index[0], loop,
                           jnp.zeros((pad_e,), jnp.int32)])
    dst = jnp.concatenate([edge_index[1], loop,
                           jnp.full((pad_e,), N, jnp.int32)])
    eye1 = jnp.eye(HEADS1, dtype=f32)
    a_s1 = (att_src1[:, :, None] * eye1[:, None, :]).reshape(D1, HEADS1)
    a_d1 = (att_dst1[:, :, None] * eye1[:, None, :]).reshape(D1, HEADS1)
    r1 = jnp.concatenate(
        [jnp.kron(jnp.eye(HEADS1, dtype=f32), jnp.ones((1, C1), f32)),
         jnp.zeros((16 - HEADS1, D1), f32)], axis=0)
    r2 = jnp.concatenate([jnp.ones((1, D_OUT), f32),
                          jnp.zeros((15, D_OUT), f32)], axis=0)
    a2 = jnp.concatenate(
        [att_src2.T, jnp.zeros((D_OUT, 7), f32),
         att_dst2.T, jnp.zeros((D_OUT, 7), f32)], axis=1)
    b1r = b1.reshape(1, D1)
    b2r = b2.reshape(1, D_OUT)

    blk = 512
    g1 = NP // blk
    w1acc = D1 + 16
    t1, aux1 = pl.pallas_call(
        _mm1_body,
        grid=(g1,),
        in_specs=[
            pl.BlockSpec((blk, D_IN), lambda i: (i, 0)),
            pl.BlockSpec((D_IN, D1), lambda i: (0, 0)),
            pl.BlockSpec((D1, HEADS1), lambda i: (0, 0)),
            pl.BlockSpec((D1, HEADS1), lambda i: (0, 0)),
        ],
        out_specs=[
            pl.BlockSpec((blk, w1acc), lambda i: (i, 0)),
            pl.BlockSpec((blk, 16), lambda i: (i, 0)),
        ],
        out_shape=[
            jax.ShapeDtypeStruct((NP, w1acc), f32),
            jax.ShapeDtypeStruct((NP, 16), f32),
        ],
    )(x, W1, a_s1, a_d1)

    acc1a, acc1b = _edge_pass1(t1, aux1, src, dst)

    w2acc = D_OUT + 16
    t2, aux2 = pl.pallas_call(
        _mid_body,
        grid=(g1,),
        in_specs=[
            pl.BlockSpec((blk, w1acc), lambda i: (i, 0)),
            pl.BlockSpec((blk, w1acc), lambda i: (i, 0)),
            pl.BlockSpec((16, D1), lambda i: (0, 0)),
            pl.BlockSpec((1, D1), lambda i: (0, 0)),
            pl.BlockSpec((D1, D_OUT), lambda i: (0, 0)),
            pl.BlockSpec((D_OUT, 16), lambda i: (0, 0)),
        ],
        out_specs=[
            pl.BlockSpec((blk, w2acc), lambda i: (i, 0)),
            pl.BlockSpec((blk, 16), lambda i: (i, 0)),
        ],
        out_shape=[
            jax.ShapeDtypeStruct((NP, w2acc), f32),
            jax.ShapeDtypeStruct((NP, 16), f32),
        ],
    )(acc1a, acc1b, r1, b1r, W2, a2)

    acc2a, acc2b = _edge_pass2(t2, aux2, src, dst)

    blk_c = 1000
    g2 = N // blk_c
    h, logp = pl.pallas_call(
        _final_body,
        grid=(g2,),
        in_specs=[
            pl.BlockSpec((blk_c, w2acc), lambda i: (i, 0)),
            pl.BlockSpec((blk_c, w2acc), lambda i: (i, 0)),
            pl.BlockSpec((16, D_OUT), lambda i: (0, 0)),
            pl.BlockSpec((1, D_OUT), lambda i: (0, 0)),
        ],
        out_specs=[
            pl.BlockSpec((blk_c, D_OUT), lambda i: (i, 0)),
            pl.BlockSpec((blk_c, D_OUT), lambda i: (i, 0)),
        ],
        out_shape=[
            jax.ShapeDtypeStruct((N, D_OUT), f32),
            jax.ShapeDtypeStruct((N, D_OUT), f32),
        ],
    )(acc2a, acc2b, r2, b2r)

    return (h, logp)


# speedup vs baseline: 79.2634x; 1.0110x over previous
import functools

import jax
import jax.numpy as jnp
from jax import lax
from jax.experimental import pallas as pl
from jax.experimental.pallas import tpu as pltpu
from jax.experimental.pallas import tpu_sc as plsc

N = 10000
D_IN = 128
HEADS1 = 8
C1 = 16
D1 = HEADS1 * C1
D_OUT = 64

NP = 10240
NC, NS, LANES = 2, 16, 16
NW = NC * NS
CH = 96
E_TOT = 320000 + N
EW = 10368
EP = NW * EW
NCHUNK = EW // CH
ROWS_PER_TILE = NP // NS


def _sc_edge_pass(feat_d, heads):
    d = feat_d
    w = d + 16
    vh = d // heads // LANES
    mesh = plsc.VectorSubcoreMesh(core_axis_name="c", subcore_axis_name="s")

    @functools.partial(
        pl.kernel,
        out_type=(jax.ShapeDtypeStruct((NP, w), jnp.float32),
                  jax.ShapeDtypeStruct((NP, w), jnp.float32)),
        mesh=mesh,
        scratch_types=[
            [pltpu.VMEM((CH,), jnp.int32)] * 2,
            [pltpu.VMEM((CH,), jnp.int32)] * 2,
            [pltpu.VMEM((CH,), jnp.int32)] * 2,
            [pltpu.VMEM((CH, w), jnp.float32)] * 2,
            [pltpu.VMEM((CH, 16), jnp.float32)] * 2,
            pltpu.VMEM((CH, 16), jnp.float32),
            pltpu.VMEM_SHARED((NP, w), jnp.float32),
            [pltpu.SemaphoreType.DMA] * 2,
            [pltpu.SemaphoreType.DMA] * 2,
            [pltpu.SemaphoreType.DMA] * 2,
            [pltpu.SemaphoreType.DMA] * 2,
            [pltpu.SemaphoreType.DMA] * 2,
            [pltpu.SemaphoreType.DMA] * 2,
        ],
        compiler_params=pltpu.CompilerParams(use_tc_tiling_on_sc=False,
                                             needs_layout_passes=False),
    )
    def edge_pass(t_hbm, aux_hbm, edges_hbm, out0, out1,
                  idx_s, idx_da, idx_db, srow, drow, ex, acc,
                  sgt, sga, sis, sia, sib, ss):
        cid = lax.axis_index("c")
        sid = lax.axis_index("s")
        zero16 = jnp.zeros((LANES,), jnp.float32)

        def zero_body(e, carry):
            for v in range(w // LANES):
                srow[0][e, pl.ds(v * LANES, LANES)] = zero16
            ex[e, pl.ds(0, LANES)] = zero16
            return carry
        lax.fori_loop(0, CH, zero_body, 0)
        tile_base = sid * ROWS_PER_TILE
        nfull = ROWS_PER_TILE // CH
        for k in range(nfull):
            pltpu.sync_copy(srow[0], acc.at[pl.ds(tile_base + k * CH, CH)])
        rem = ROWS_PER_TILE - nfull * CH
        if rem:
            pltpu.sync_copy(
                srow[0].at[pl.ds(0, rem)],
                acc.at[pl.ds(tile_base + nfull * CH, rem)])
        plsc.subcore_barrier()

        base_edges = (cid * NS + sid) * EW

        def load_idx_s(j, b):
            pltpu.async_copy(
                edges_hbm.at[pl.ds(base_edges + j * CH, CH)], idx_s[b], sis[b])

        def load_idx_da(j, b):
            pltpu.async_copy(
                edges_hbm.at[pl.ds(EP + base_edges + j * CH, CH)], idx_da[b],
                sia[b])

        def load_idx_db(j, b):
            pltpu.async_copy(
                edges_hbm.at[pl.ds(EP + base_edges + j * CH, CH)], idx_db[b],
                sib[b])

        def issue_gathers(b):
            pltpu.async_copy(t_hbm.at[idx_s[b]], srow[b], sgt[b])
            pltpu.async_copy(aux_hbm.at[idx_da[b]], drow[b], sga[b])

        def wait_gathers(b):
            pltpu.make_async_copy(t_hbm.at[idx_s[b]], srow[b], sgt[b]).wait()
            pltpu.make_async_copy(aux_hbm.at[idx_da[b]], drow[b],
                                  sga[b]).wait()

        def wait_idx_s(b):
            pltpu.make_async_copy(
                edges_hbm.at[pl.ds(0, CH)], idx_s[b], sis[b]).wait()

        def wait_idx_da(b):
            pltpu.make_async_copy(
                edges_hbm.at[pl.ds(0, CH)], idx_da[b], sia[b]).wait()

        def wait_idx_db(b):
            pltpu.make_async_copy(
                edges_hbm.at[pl.ds(0, CH)], idx_db[b], sib[b]).wait()

        def wait_scatter(b):
            pltpu.make_async_copy(srow[b], acc.at[idx_db[b]], ss[b]).wait()

        def compute(b):
            sr, dr = srow[b], drow[b]
            for g in range(CH // LANES):
                rows = lax.iota(jnp.int32, LANES) + g * LANES
                for h in range(heads):
                    vs = plsc.load_gather(
                        sr, [rows, jnp.full((LANES,), d + h, jnp.int32)])
                    vd = plsc.load_gather(
                        dr, [rows, jnp.full((LANES,), 8 + h, jnp.int32)])
                    al = vs + vd
                    al = jnp.where(al >= 0.0, al, 0.2 * al)
                    plsc.store_scatter(
                        ex, [rows, jnp.full((LANES,), h, jnp.int32)],
                        jnp.exp(al))

            def msg_body(i, carry):
                e0 = i * 2
                e1 = e0 + 1
                exv0 = ex[e0, pl.ds(0, LANES)]
                exv1 = ex[e1, pl.ds(0, LANES)]
                sr[e0, pl.ds(d, LANES)] = exv0
                sr[e1, pl.ds(d, LANES)] = exv1
                for h in range(heads):
                    s0 = exv0[h]
                    s1 = exv1[h]
                    for v in range(vh):
                        col = (h * vh + v) * LANES
                        sr[e0, pl.ds(col, LANES)] = (
                            sr[e0, pl.ds(col, LANES)] * s0)
                        sr[e1, pl.ds(col, LANES)] = (
                            sr[e1, pl.ds(col, LANES)] * s1)
                return carry
            lax.fori_loop(0, CH // 2, msg_body, 0)

        def step(j, b, nb):
            wait_gathers(b)

            @pl.when(j + 2 < NCHUNK)
            def _():
                load_idx_s(j + 2, b)
                load_idx_da(j + 2, b)

            @pl.when(j + 1 < NCHUNK)
            def _():
                wait_idx_s(nb)
                wait_idx_da(nb)

                @pl.when(j >= 1)
                def _():
                    wait_scatter(nb)
                load_idx_db(j + 1, nb)
                issue_gathers(nb)
            compute(b)

            @pl.when(j >= 1)
            def _():
                wait_idx_db(b)
            pltpu.async_copy(srow[b], acc.at[idx_db[b]], ss[b], add=True)

        pltpu.sync_copy(edges_hbm.at[pl.ds(base_edges, CH)], idx_s[0])
        pltpu.sync_copy(edges_hbm.at[pl.ds(EP + base_edges, CH)], idx_da[0])
        pltpu.sync_copy(edges_hbm.at[pl.ds(EP + base_edges, CH)], idx_db[0])
        issue_gathers(0)
        load_idx_s(1, 1)
        load_idx_da(1, 1)

        def pair_body(j2, carry):
            j = j2 * 2
            step(j, 0, 1)
            step(j + 1, 1, 0)
            return carry
        lax.fori_loop(0, NCHUNK // 2, pair_body, 0)
        wait_scatter(0)
        wait_scatter(1)

        plsc.subcore_barrier()
        my_rows = pl.ds(tile_base, ROWS_PER_TILE)

        @pl.when(cid == 0)
        def _():
            pltpu.sync_copy(acc.at[my_rows], out0.at[my_rows])

        @pl.when(cid == 1)
        def _():
            pltpu.sync_copy(acc.at[my_rows], out1.at[my_rows])

    return edge_pass


_edge_pass1 = _sc_edge_pass(D1, HEADS1)
_edge_pass2 = _sc_edge_pass(D_OUT, 1)


def _mm1_body(x_ref, w_ref, as_ref, ad_ref, t1_ref, aux1_ref):
    xw = jnp.dot(x_ref[...], w_ref[...], preferred_element_type=jnp.float32)
    asrc = jnp.dot(xw, as_ref[...], preferred_element_type=jnp.float32)
    adst = jnp.dot(xw, ad_ref[...], preferred_element_type=jnp.float32)
    aux = jnp.concatenate([asrc, adst], axis=1)
    t1_ref[...] = jnp.concatenate([xw, aux], axis=1)
    aux1_ref[...] = aux


def _mid_body(a0_ref, a1_ref, r1_ref, b1_ref, w2_ref, a2_ref,
              t2_ref, aux2_ref):
    acc = a0_ref[...] + a1_ref[...]
    num = acc[:, :D1]
    den = jnp.dot(acc[:, D1:], r1_ref[...], preferred_element_type=jnp.float32)
    h1 = num / (den + 1e-16) + b1_ref[...]
    h1 = jnp.where(h1 > 0.0, h1, jnp.exp(h1) - 1.0)
    h2 = jnp.dot(h1, w2_ref[...], preferred_element_type=jnp.float32)
    aux2 = jnp.dot(h2, a2_ref[...], preferred_element_type=jnp.float32)
    t2_ref[...] = jnp.concatenate([h2, aux2], axis=1)
    aux2_ref[...] = aux2


def _final_body(a0_ref, a1_ref, r2_ref, b2_ref, h_ref, lp_ref):
    acc = a0_ref[...] + a1_ref[...]
    num = acc[:, :D_OUT]
    den = jnp.dot(acc[:, D_OUT:], r2_ref[...],
                  preferred_element_type=jnp.float32)
    h = num / (den + 1e-16) + b2_ref[...]
    h_ref[...] = h
    m = jnp.max(h, axis=1, keepdims=True)
    lse = jnp.log(jnp.sum(jnp.exp(h - m), axis=1, keepdims=True)) + m
    lp_ref[...] = h - lse


def kernel(x, edge_index, W1, att_src1, att_dst1, b1, W2, att_src2,
           att_dst2, b2):
    f32 = jnp.float32
    loop = jnp.arange(N, dtype=jnp.int32)
    pad_e = EP - E_TOT
    edges = jnp.concatenate([
        edge_index[0], loop, jnp.zeros((pad_e,), jnp.int32),
        edge_index[1], loop, jnp.full((pad_e,), N, jnp.int32)])
    eye1 = jnp.eye(HEADS1, dtype=f32)
    a_s1 = (att_src1[:, :, None] * eye1[:, None, :]).reshape(D1, HEADS1)
    a_d1 = (att_dst1[:, :, None] * eye1[:, None, :]).reshape(D1, HEADS1)
    r1 = jnp.concatenate(
        [jnp.kron(jnp.eye(HEADS1, dtype=f32), jnp.ones((1, C1), f32)),
         jnp.zeros((16 - HEADS1, D1), f32)], axis=0)
    r2 = jnp.concatenate([jnp.ones((1, D_OUT), f32),
                          jnp.zeros((15, D_OUT), f32)], axis=0)
    a2 = jnp.concatenate(
        [att_src2.T, jnp.zeros((D_OUT, 7), f32),
         att_dst2.T, jnp.zeros((D_OUT, 7), f32)], axis=1)
    b1r = b1.reshape(1, D1)
    b2r = b2.reshape(1, D_OUT)

    blk = 512
    g1 = NP // blk
    w1acc = D1 + 16
    t1, aux1 = pl.pallas_call(
        _mm1_body,
        grid=(g1,),
        in_specs=[
            pl.BlockSpec((blk, D_IN), lambda i: (i, 0)),
            pl.BlockSpec((D_IN, D1), lambda i: (0, 0)),
            pl.BlockSpec((D1, HEADS1), lambda i: (0, 0)),
            pl.BlockSpec((D1, HEADS1), lambda i: (0, 0)),
        ],
        out_specs=[
            pl.BlockSpec((blk, w1acc), lambda i: (i, 0)),
            pl.BlockSpec((blk, 16), lambda i: (i, 0)),
        ],
        out_shape=[
            jax.ShapeDtypeStruct((NP, w1acc), f32),
            jax.ShapeDtypeStruct((NP, 16), f32),
        ],
    )(x, W1, a_s1, a_d1)

    acc1a, acc1b = _edge_pass1(t1, aux1, edges)

    w2acc = D_OUT + 16
    t2, aux2 = pl.pallas_call(
        _mid_body,
        grid=(g1,),
        in_specs=[
            pl.BlockSpec((blk, w1acc), lambda i: (i, 0)),
            pl.BlockSpec((blk, w1acc), lambda i: (i, 0)),
            pl.BlockSpec((16, D1), lambda i: (0, 0)),
            pl.BlockSpec((1, D1), lambda i: (0, 0)),
            pl.BlockSpec((D1, D_OUT), lambda i: (0, 0)),
            pl.BlockSpec((D_OUT, 16), lambda i: (0, 0)),
        ],
        out_specs=[
            pl.BlockSpec((blk, w2acc), lambda i: (i, 0)),
            pl.BlockSpec((blk, 16), lambda i: (i, 0)),
        ],
        out_shape=[
            jax.ShapeDtypeStruct((NP, w2acc), f32),
            jax.ShapeDtypeStruct((NP, 16), f32),
        ],
    )(acc1a, acc1b, r1, b1r, W2, a2)

    acc2a, acc2b = _edge_pass2(t2, aux2, edges)

    blk_c = 1000
    g2 = N // blk_c
    h, logp = pl.pallas_call(
        _final_body,
        grid=(g2,),
        in_specs=[
            pl.BlockSpec((blk_c, w2acc), lambda i: (i, 0)),
            pl.BlockSpec((blk_c, w2acc), lambda i: (i, 0)),
            pl.BlockSpec((16, D_OUT), lambda i: (0, 0)),
            pl.BlockSpec((1, D_OUT), lambda i: (0, 0)),
        ],
        out_specs=[
            pl.BlockSpec((blk_c, D_OUT), lambda i: (i, 0)),
            pl.BlockSpec((blk_c, D_OUT), lambda i: (i, 0)),
        ],
        out_shape=[
            jax.ShapeDtypeStruct((N, D_OUT), f32),
            jax.ShapeDtypeStruct((N, D_OUT), f32),
        ],
    )(acc2a, acc2b, r2, b2r)

    return (h, logp)


# speedup vs baseline: 82.3162x; 1.0385x over previous
import functools

import jax
import jax.numpy as jnp
from jax import lax
from jax.experimental import pallas as pl
from jax.experimental.pallas import tpu as pltpu
from jax.experimental.pallas import tpu_sc as plsc

N = 10000
D_IN = 128
HEADS1 = 8
C1 = 16
D1 = HEADS1 * C1
D_OUT = 64

NP = 10240
NC, NS, LANES = 2, 16, 16
NW = NC * NS
CH = 96
E_TOT = 320000 + N
EP = 331776
NCHUNK0 = 116
NCHUNK1 = 100
ROWS_PER_TILE = NP // NS


def _sc_edge_pass(feat_d, heads):
    d = feat_d
    w = d + 16
    vh = d // heads // LANES
    mesh = plsc.VectorSubcoreMesh(core_axis_name="c", subcore_axis_name="s")

    @functools.partial(
        pl.kernel,
        out_type=(jax.ShapeDtypeStruct((NP, w), jnp.float32),
                  jax.ShapeDtypeStruct((NP, w), jnp.float32)),
        mesh=mesh,
        scratch_types=[
            [pltpu.VMEM((CH,), jnp.int32)] * 2,
            [pltpu.VMEM((CH,), jnp.int32)] * 2,
            [pltpu.VMEM((CH,), jnp.int32)] * 2,
            [pltpu.VMEM((CH, w), jnp.float32)] * 2,
            [pltpu.VMEM((CH, 16), jnp.float32)] * 2,
            pltpu.VMEM((CH, 16), jnp.float32),
            pltpu.VMEM_SHARED((NP, w), jnp.float32),
            [pltpu.SemaphoreType.DMA] * 2,
            [pltpu.SemaphoreType.DMA] * 2,
            [pltpu.SemaphoreType.DMA] * 2,
            [pltpu.SemaphoreType.DMA] * 2,
            [pltpu.SemaphoreType.DMA] * 2,
            [pltpu.SemaphoreType.DMA] * 2,
        ],
        compiler_params=pltpu.CompilerParams(use_tc_tiling_on_sc=False,
                                             needs_layout_passes=False),
    )
    def edge_pass(t_hbm, aux_hbm, edges_hbm, out0, out1,
                  idx_s, idx_da, idx_db, srow, drow, ex, acc,
                  sgt, sga, sis, sia, sib, ss):
        cid = lax.axis_index("c")
        sid = lax.axis_index("s")
        zero16 = jnp.zeros((LANES,), jnp.float32)

        def zero_body(e, carry):
            for v in range(w // LANES):
                srow[0][e, pl.ds(v * LANES, LANES)] = zero16
            ex[e, pl.ds(0, LANES)] = zero16
            return carry
        lax.fori_loop(0, CH, zero_body, 0)
        tile_base = sid * ROWS_PER_TILE
        nfull = ROWS_PER_TILE // CH
        for k in range(nfull):
            pltpu.sync_copy(srow[0], acc.at[pl.ds(tile_base + k * CH, CH)])
        rem = ROWS_PER_TILE - nfull * CH
        if rem:
            pltpu.sync_copy(
                srow[0].at[pl.ds(0, rem)],
                acc.at[pl.ds(tile_base + nfull * CH, rem)])
        plsc.subcore_barrier()

        nchunk = jnp.where(cid == 0, NCHUNK0, NCHUNK1)
        ew_c = jnp.where(cid == 0, NCHUNK0 * CH, NCHUNK1 * CH)
        base_edges = cid * (NS * NCHUNK0 * CH) + sid * ew_c

        def load_idx_s(j, b):
            pltpu.async_copy(
                edges_hbm.at[pl.ds(base_edges + j * CH, CH)], idx_s[b], sis[b])

        def load_idx_da(j, b):
            pltpu.async_copy(
                edges_hbm.at[pl.ds(EP + base_edges + j * CH, CH)], idx_da[b],
                sia[b])

        def load_idx_db(j, b):
            pltpu.async_copy(
                edges_hbm.at[pl.ds(EP + base_edges + j * CH, CH)], idx_db[b],
                sib[b])

        def issue_gathers(b):
            pltpu.async_copy(t_hbm.at[idx_s[b]], srow[b], sgt[b])
            pltpu.async_copy(aux_hbm.at[idx_da[b]], drow[b], sga[b])

        def wait_gathers(b):
            pltpu.make_async_copy(t_hbm.at[idx_s[b]], srow[b], sgt[b]).wait()
            pltpu.make_async_copy(aux_hbm.at[idx_da[b]], drow[b],
                                  sga[b]).wait()

        def wait_idx_s(b):
            pltpu.make_async_copy(
                edges_hbm.at[pl.ds(0, CH)], idx_s[b], sis[b]).wait()

        def wait_idx_da(b):
            pltpu.make_async_copy(
                edges_hbm.at[pl.ds(0, CH)], idx_da[b], sia[b]).wait()

        def wait_idx_db(b):
            pltpu.make_async_copy(
                edges_hbm.at[pl.ds(0, CH)], idx_db[b], sib[b]).wait()

        def wait_scatter(b):
            pltpu.make_async_copy(srow[b], acc.at[idx_db[b]], ss[b]).wait()

        def compute(b):
            sr, dr = srow[b], drow[b]
            for g in range(CH // LANES):
                rows = lax.iota(jnp.int32, LANES) + g * LANES
                for h in range(heads):
                    vs = plsc.load_gather(
                        sr, [rows, jnp.full((LANES,), d + h, jnp.int32)])
                    vd = plsc.load_gather(
                        dr, [rows, jnp.full((LANES,), 8 + h, jnp.int32)])
                    al = vs + vd
                    al = jnp.where(al >= 0.0, al, 0.2 * al)
                    plsc.store_scatter(
                        ex, [rows, jnp.full((LANES,), h, jnp.int32)],
                        jnp.exp(al))

            def msg_body(i, carry):
                e0 = i * 2
                e1 = e0 + 1
                exv0 = ex[e0, pl.ds(0, LANES)]
                exv1 = ex[e1, pl.ds(0, LANES)]
                sr[e0, pl.ds(d, LANES)] = exv0
                sr[e1, pl.ds(d, LANES)] = exv1
                for h in range(heads):
                    s0 = exv0[h]
                    s1 = exv1[h]
                    for v in range(vh):
                        col = (h * vh + v) * LANES
                        sr[e0, pl.ds(col, LANES)] = (
                            sr[e0, pl.ds(col, LANES)] * s0)
                        sr[e1, pl.ds(col, LANES)] = (
                            sr[e1, pl.ds(col, LANES)] * s1)
                return carry
            lax.fori_loop(0, CH // 2, msg_body, 0)

        def step(j, b, nb):
            wait_gathers(b)

            @pl.when(j + 2 < nchunk)
            def _():
                load_idx_s(j + 2, b)
                load_idx_da(j + 2, b)

            @pl.when(j + 1 < nchunk)
            def _():
                wait_idx_s(nb)
                wait_idx_da(nb)

                @pl.when(j >= 1)
                def _():
                    wait_scatter(nb)
                load_idx_db(j + 1, nb)
                issue_gathers(nb)
            compute(b)

            @pl.when(j >= 1)
            def _():
                wait_idx_db(b)
            pltpu.async_copy(srow[b], acc.at[idx_db[b]], ss[b], add=True)

        pltpu.sync_copy(edges_hbm.at[pl.ds(base_edges, CH)], idx_s[0])
        pltpu.sync_copy(edges_hbm.at[pl.ds(EP + base_edges, CH)], idx_da[0])
        pltpu.sync_copy(edges_hbm.at[pl.ds(EP + base_edges, CH)], idx_db[0])
        issue_gathers(0)
        load_idx_s(1, 1)
        load_idx_da(1, 1)

        def pair_body(j2, carry):
            j = j2 * 2
            step(j, 0, 1)
            step(j + 1, 1, 0)
            return carry
        lax.fori_loop(0, nchunk // 2, pair_body, 0)
        wait_scatter(0)
        wait_scatter(1)

        plsc.subcore_barrier()
        my_rows = pl.ds(tile_base, ROWS_PER_TILE)

        @pl.when(cid == 0)
        def _():
            pltpu.sync_copy(acc.at[my_rows], out0.at[my_rows])

        @pl.when(cid == 1)
        def _():
            pltpu.sync_copy(acc.at[my_rows], out1.at[my_rows])

    return edge_pass


_edge_pass1 = _sc_edge_pass(D1, HEADS1)
_edge_pass2 = _sc_edge_pass(D_OUT, 1)


def _mm1_body(x_ref, w_ref, as_ref, ad_ref, t1_ref, aux1_ref):
    xw = jnp.dot(x_ref[...], w_ref[...], preferred_element_type=jnp.float32)
    asrc = jnp.dot(xw, as_ref[...], preferred_element_type=jnp.float32)
    adst = jnp.dot(xw, ad_ref[...], preferred_element_type=jnp.float32)
    aux = jnp.concatenate([asrc, adst], axis=1)
    t1_ref[...] = jnp.concatenate([xw, aux], axis=1)
    aux1_ref[...] = aux


def _mid_body(a0_ref, a1_ref, r1_ref, b1_ref, w2_ref, a2_ref,
              t2_ref, aux2_ref):
    acc = a0_ref[...] + a1_ref[...]
    num = acc[:, :D1]
    den = jnp.dot(acc[:, D1:], r1_ref[...], preferred_element_type=jnp.float32)
    h1 = num / (den + 1e-16) + b1_ref[...]
    h1 = jnp.where(h1 > 0.0, h1, jnp.exp(h1) - 1.0)
    h2 = jnp.dot(h1, w2_ref[...], preferred_element_type=jnp.float32)
    aux2 = jnp.dot(h2, a2_ref[...], preferred_element_type=jnp.float32)
    t2_ref[...] = jnp.concatenate([h2, aux2], axis=1)
    aux2_ref[...] = aux2


def _final_body(a0_ref, a1_ref, r2_ref, b2_ref, h_ref, lp_ref):
    acc = a0_ref[...] + a1_ref[...]
    num = acc[:, :D_OUT]
    den = jnp.dot(acc[:, D_OUT:], r2_ref[...],
                  preferred_element_type=jnp.float32)
    h = num / (den + 1e-16) + b2_ref[...]
    h_ref[...] = h
    m = jnp.max(h, axis=1, keepdims=True)
    lse = jnp.log(jnp.sum(jnp.exp(h - m), axis=1, keepdims=True)) + m
    lp_ref[...] = h - lse


def kernel(x, edge_index, W1, att_src1, att_dst1, b1, W2, att_src2,
           att_dst2, b2):
    f32 = jnp.float32
    loop = jnp.arange(N, dtype=jnp.int32)
    pad_e = EP - E_TOT
    edges = jnp.concatenate([
        edge_index[0], loop, jnp.zeros((pad_e,), jnp.int32),
        edge_index[1], loop, jnp.full((pad_e,), N, jnp.int32)])
    eye1 = jnp.eye(HEADS1, dtype=f32)
    a_s1 = (att_src1[:, :, None] * eye1[:, None, :]).reshape(D1, HEADS1)
    a_d1 = (att_dst1[:, :, None] * eye1[:, None, :]).reshape(D1, HEADS1)
    r1 = jnp.concatenate(
        [jnp.kron(jnp.eye(HEADS1, dtype=f32), jnp.ones((1, C1), f32)),
         jnp.zeros((16 - HEADS1, D1), f32)], axis=0)
    r2 = jnp.concatenate([jnp.ones((1, D_OUT), f32),
                          jnp.zeros((15, D_OUT), f32)], axis=0)
    a2 = jnp.concatenate(
        [att_src2.T, jnp.zeros((D_OUT, 7), f32),
         att_dst2.T, jnp.zeros((D_OUT, 7), f32)], axis=1)
    b1r = b1.reshape(1, D1)
    b2r = b2.reshape(1, D_OUT)

    blk = 512
    g1 = NP // blk
    w1acc = D1 + 16
    t1, aux1 = pl.pallas_call(
        _mm1_body,
        grid=(g1,),
        in_specs=[
            pl.BlockSpec((blk, D_IN), lambda i: (i, 0)),
            pl.BlockSpec((D_IN, D1), lambda i: (0, 0)),
            pl.BlockSpec((D1, HEADS1), lambda i: (0, 0)),
            pl.BlockSpec((D1, HEADS1), lambda i: (0, 0)),
        ],
        out_specs=[
            pl.BlockSpec((blk, w1acc), lambda i: (i, 0)),
            pl.BlockSpec((blk, 16), lambda i: (i, 0)),
        ],
        out_shape=[
            jax.ShapeDtypeStruct((NP, w1acc), f32),
            jax.ShapeDtypeStruct((NP, 16), f32),
        ],
    )(x, W1, a_s1, a_d1)

    acc1a, acc1b = _edge_pass1(t1, aux1, edges)

    w2acc = D_OUT + 16
    t2, aux2 = pl.pallas_call(
        _mid_body,
        grid=(g1,),
        in_specs=[
            pl.BlockSpec((blk, w1acc), lambda i: (i, 0)),
            pl.BlockSpec((blk, w1acc), lambda i: (i, 0)),
            pl.BlockSpec((16, D1), lambda i: (0, 0)),
            pl.BlockSpec((1, D1), lambda i: (0, 0)),
            pl.BlockSpec((D1, D_OUT), lambda i: (0, 0)),
            pl.BlockSpec((D_OUT, 16), lambda i: (0, 0)),
        ],
        out_specs=[
            pl.BlockSpec((blk, w2acc), lambda i: (i, 0)),
            pl.BlockSpec((blk, 16), lambda i: (i, 0)),
        ],
        out_shape=[
            jax.ShapeDtypeStruct((NP, w2acc), f32),
            jax.ShapeDtypeStruct((NP, 16), f32),
        ],
    )(acc1a, acc1b, r1, b1r, W2, a2)

    acc2a, acc2b = _edge_pass2(t2, aux2, edges)

    blk_c = 1000
    g2 = N // blk_c
    h, logp = pl.pallas_call(
        _final_body,
        grid=(g2,),
        in_specs=[
            pl.BlockSpec((blk_c, w2acc), lambda i: (i, 0)),
            pl.BlockSpec((blk_c, w2acc), lambda i: (i, 0)),
            pl.BlockSpec((16, D_OUT), lambda i: (0, 0)),
            pl.BlockSpec((1, D_OUT), lambda i: (0, 0)),
        ],
        out_specs=[
            pl.BlockSpec((blk_c, D_OUT), lambda i: (i, 0)),
            pl.BlockSpec((blk_c, D_OUT), lambda i: (i, 0)),
        ],
        out_shape=[
            jax.ShapeDtypeStruct((N, D_OUT), f32),
            jax.ShapeDtypeStruct((N, D_OUT), f32),
        ],
    )(acc2a, acc2b, r2, b2r)

    return (h, logp)


# speedup vs baseline: 84.0888x; 1.0215x over previous
import functools

import jax
import jax.numpy as jnp
from jax import lax
from jax.experimental import pallas as pl
from jax.experimental.pallas import tpu as pltpu
from jax.experimental.pallas import tpu_sc as plsc

N = 10000
D_IN = 128
HEADS1 = 8
C1 = 16
D1 = HEADS1 * C1
D_OUT = 64

NP = 10240
NC, NS, LANES = 2, 16, 16
NW = NC * NS
CH = 96
E_TOT = 320000 + N
EP = 331776
NCHUNK0 = 120
NCHUNK1 = 96
ROWS_PER_TILE = NP // NS


def _sc_edge_pass(feat_d, heads):
    d = feat_d
    w = d + 16
    vh = d // heads // LANES
    mesh = plsc.VectorSubcoreMesh(core_axis_name="c", subcore_axis_name="s")

    @functools.partial(
        pl.kernel,
        out_type=(jax.ShapeDtypeStruct((NP, w), jnp.float32),
                  jax.ShapeDtypeStruct((NP, w), jnp.float32)),
        mesh=mesh,
        scratch_types=[
            [pltpu.VMEM((CH,), jnp.int32)] * 2,
            [pltpu.VMEM((CH,), jnp.int32)] * 2,
            [pltpu.VMEM((CH,), jnp.int32)] * 2,
            [pltpu.VMEM((CH, w), jnp.float32)] * 2,
            [pltpu.VMEM((CH, 16), jnp.float32)] * 2,
            pltpu.VMEM((CH, 16), jnp.float32),
            pltpu.VMEM_SHARED((NP, w), jnp.float32),
            [pltpu.SemaphoreType.DMA] * 2,
            [pltpu.SemaphoreType.DMA] * 2,
            [pltpu.SemaphoreType.DMA] * 2,
            [pltpu.SemaphoreType.DMA] * 2,
            [pltpu.SemaphoreType.DMA] * 2,
            [pltpu.SemaphoreType.DMA] * 2,
        ],
        compiler_params=pltpu.CompilerParams(use_tc_tiling_on_sc=False,
                                             needs_layout_passes=False),
    )
    def edge_pass(t_hbm, aux_hbm, edges_hbm, out0, out1,
                  idx_s, idx_da, idx_db, srow, drow, ex, acc,
                  sgt, sga, sis, sia, sib, ss):
        cid = lax.axis_index("c")
        sid = lax.axis_index("s")
        zero16 = jnp.zeros((LANES,), jnp.float32)

        def zero_body(e, carry):
            for v in range(w // LANES):
                srow[0][e, pl.ds(v * LANES, LANES)] = zero16
            ex[e, pl.ds(0, LANES)] = zero16
            return carry
        lax.fori_loop(0, CH, zero_body, 0)
        tile_base = sid * ROWS_PER_TILE
        nfull = ROWS_PER_TILE // CH
        for k in range(nfull):
            pltpu.sync_copy(srow[0], acc.at[pl.ds(tile_base + k * CH, CH)])
        rem = ROWS_PER_TILE - nfull * CH
        if rem:
            pltpu.sync_copy(
                srow[0].at[pl.ds(0, rem)],
                acc.at[pl.ds(tile_base + nfull * CH, rem)])
        plsc.subcore_barrier()

        nchunk = jnp.where(cid == 0, NCHUNK0, NCHUNK1)
        ew_c = jnp.where(cid == 0, NCHUNK0 * CH, NCHUNK1 * CH)
        base_edges = cid * (NS * NCHUNK0 * CH) + sid * ew_c

        def load_idx_s(j, b):
            pltpu.async_copy(
                edges_hbm.at[pl.ds(base_edges + j * CH, CH)], idx_s[b], sis[b])

        def load_idx_da(j, b):
            pltpu.async_copy(
                edges_hbm.at[pl.ds(EP + base_edges + j * CH, CH)], idx_da[b],
                sia[b])

        def load_idx_db(j, b):
            pltpu.async_copy(
                edges_hbm.at[pl.ds(EP + base_edges + j * CH, CH)], idx_db[b],
                sib[b])

        def issue_gathers(b):
            pltpu.async_copy(t_hbm.at[idx_s[b]], srow[b], sgt[b])
            pltpu.async_copy(aux_hbm.at[idx_da[b]], drow[b], sga[b])

        def wait_gathers(b):
            pltpu.make_async_copy(t_hbm.at[idx_s[b]], srow[b], sgt[b]).wait()
            pltpu.make_async_copy(aux_hbm.at[idx_da[b]], drow[b],
                                  sga[b]).wait()

        def wait_idx_s(b):
            pltpu.make_async_copy(
                edges_hbm.at[pl.ds(0, CH)], idx_s[b], sis[b]).wait()

        def wait_idx_da(b):
            pltpu.make_async_copy(
                edges_hbm.at[pl.ds(0, CH)], idx_da[b], sia[b]).wait()

        def wait_idx_db(b):
            pltpu.make_async_copy(
                edges_hbm.at[pl.ds(0, CH)], idx_db[b], sib[b]).wait()

        def wait_scatter(b):
            pltpu.make_async_copy(srow[b], acc.at[idx_db[b]], ss[b]).wait()

        def compute(b):
            sr, dr = srow[b], drow[b]
            for g in range(CH // LANES):
                rows = lax.iota(jnp.int32, LANES) + g * LANES
                for h in range(heads):
                    vs = plsc.load_gather(
                        sr, [rows, jnp.full((LANES,), d + h, jnp.int32)])
                    vd = plsc.load_gather(
                        dr, [rows, jnp.full((LANES,), 8 + h, jnp.int32)])
                    al = vs + vd
                    al = jnp.where(al >= 0.0, al, 0.2 * al)
                    plsc.store_scatter(
                        ex, [rows, jnp.full((LANES,), h, jnp.int32)],
                        jnp.exp(al))

            def msg_body(i, carry):
                e0 = i * 2
                e1 = e0 + 1
                exv0 = ex[e0, pl.ds(0, LANES)]
                exv1 = ex[e1, pl.ds(0, LANES)]
                sr[e0, pl.ds(d, LANES)] = exv0
                sr[e1, pl.ds(d, LANES)] = exv1
                for h in range(heads):
                    s0 = exv0[h]
                    s1 = exv1[h]
                    for v in range(vh):
                        col = (h * vh + v) * LANES
                        sr[e0, pl.ds(col, LANES)] = (
                            sr[e0, pl.ds(col, LANES)] * s0)
                        sr[e1, pl.ds(col, LANES)] = (
                            sr[e1, pl.ds(col, LANES)] * s1)
                return carry
            lax.fori_loop(0, CH // 2, msg_body, 0)

        def step(j, b, nb):
            wait_gathers(b)

            @pl.when(j + 2 < nchunk)
            def _():
                load_idx_s(j + 2, b)
                load_idx_da(j + 2, b)

            @pl.when(j + 1 < nchunk)
            def _():
                wait_idx_s(nb)
                wait_idx_da(nb)

                @pl.when(j >= 1)
                def _():
                    wait_scatter(nb)
                load_idx_db(j + 1, nb)
                issue_gathers(nb)
            compute(b)

            @pl.when(j >= 1)
            def _():
                wait_idx_db(b)
            pltpu.async_copy(srow[b], acc.at[idx_db[b]], ss[b], add=True)

        pltpu.sync_copy(edges_hbm.at[pl.ds(base_edges, CH)], idx_s[0])
        pltpu.sync_copy(edges_hbm.at[pl.ds(EP + base_edges, CH)], idx_da[0])
        pltpu.sync_copy(edges_hbm.at[pl.ds(EP + base_edges, CH)], idx_db[0])
        issue_gathers(0)
        load_idx_s(1, 1)
        load_idx_da(1, 1)

        def pair_body(j2, carry):
            j = j2 * 2
            step(j, 0, 1)
            step(j + 1, 1, 0)
            return carry
        lax.fori_loop(0, nchunk // 2, pair_body, 0)
        wait_scatter(0)
        wait_scatter(1)

        plsc.subcore_barrier()
        my_rows = pl.ds(tile_base, ROWS_PER_TILE)

        @pl.when(cid == 0)
        def _():
            pltpu.sync_copy(acc.at[my_rows], out0.at[my_rows])

        @pl.when(cid == 1)
        def _():
            pltpu.sync_copy(acc.at[my_rows], out1.at[my_rows])

    return edge_pass


_edge_pass1 = _sc_edge_pass(D1, HEADS1)
_edge_pass2 = _sc_edge_pass(D_OUT, 1)


def _mm1_body(x_ref, w_ref, as_ref, ad_ref, t1_ref, aux1_ref):
    xw = jnp.dot(x_ref[...], w_ref[...], preferred_element_type=jnp.float32)
    asrc = jnp.dot(xw, as_ref[...], preferred_element_type=jnp.float32)
    adst = jnp.dot(xw, ad_ref[...], preferred_element_type=jnp.float32)
    aux = jnp.concatenate([asrc, adst], axis=1)
    t1_ref[...] = jnp.concatenate([xw, aux], axis=1)
    aux1_ref[...] = aux


def _mid_body(a0_ref, a1_ref, r1_ref, b1_ref, w2_ref, a2_ref,
              t2_ref, aux2_ref):
    acc = a0_ref[...] + a1_ref[...]
    num = acc[:, :D1]
    den = jnp.dot(acc[:, D1:], r1_ref[...], preferred_element_type=jnp.float32)
    h1 = num / (den + 1e-16) + b1_ref[...]
    h1 = jnp.where(h1 > 0.0, h1, jnp.exp(h1) - 1.0)
    h2 = jnp.dot(h1, w2_ref[...], preferred_element_type=jnp.float32)
    aux2 = jnp.dot(h2, a2_ref[...], preferred_element_type=jnp.float32)
    t2_ref[...] = jnp.concatenate([h2, aux2], axis=1)
    aux2_ref[...] = aux2


def _final_body(a0_ref, a1_ref, r2_ref, b2_ref, h_ref, lp_ref):
    acc = a0_ref[...] + a1_ref[...]
    num = acc[:, :D_OUT]
    den = jnp.dot(acc[:, D_OUT:], r2_ref[...],
                  preferred_element_type=jnp.float32)
    h = num / (den + 1e-16) + b2_ref[...]
    h_ref[...] = h
    m = jnp.max(h, axis=1, keepdims=True)
    lse = jnp.log(jnp.sum(jnp.exp(h - m), axis=1, keepdims=True)) + m
    lp_ref[...] = h - lse


def kernel(x, edge_index, W1, att_src1, att_dst1, b1, W2, att_src2,
           att_dst2, b2):
    f32 = jnp.float32
    loop = jnp.arange(N, dtype=jnp.int32)
    pad_e = EP - E_TOT
    edges = jnp.concatenate([
        edge_index[0], loop, jnp.zeros((pad_e,), jnp.int32),
        edge_index[1], loop, jnp.full((pad_e,), N, jnp.int32)])
    eye1 = jnp.eye(HEADS1, dtype=f32)
    a_s1 = (att_src1[:, :, None] * eye1[:, None, :]).reshape(D1, HEADS1)
    a_d1 = (att_dst1[:, :, None] * eye1[:, None, :]).reshape(D1, HEADS1)
    r1 = jnp.concatenate(
        [jnp.kron(jnp.eye(HEADS1, dtype=f32), jnp.ones((1, C1), f32)),
         jnp.zeros((16 - HEADS1, D1), f32)], axis=0)
    r2 = jnp.concatenate([jnp.ones((1, D_OUT), f32),
                          jnp.zeros((15, D_OUT), f32)], axis=0)
    a2 = jnp.concatenate(
        [att_src2.T, jnp.zeros((D_OUT, 7), f32),
         att_dst2.T, jnp.zeros((D_OUT, 7), f32)], axis=1)
    b1r = b1.reshape(1, D1)
    b2r = b2.reshape(1, D_OUT)

    blk = 512
    g1 = NP // blk
    w1acc = D1 + 16
    t1, aux1 = pl.pallas_call(
        _mm1_body,
        grid=(g1,),
        in_specs=[
            pl.BlockSpec((blk, D_IN), lambda i: (i, 0)),
            pl.BlockSpec((D_IN, D1), lambda i: (0, 0)),
            pl.BlockSpec((D1, HEADS1), lambda i: (0, 0)),
            pl.BlockSpec((D1, HEADS1), lambda i: (0, 0)),
        ],
        out_specs=[
            pl.BlockSpec((blk, w1acc), lambda i: (i, 0)),
            pl.BlockSpec((blk, 16), lambda i: (i, 0)),
        ],
        out_shape=[
            jax.ShapeDtypeStruct((NP, w1acc), f32),
            jax.ShapeDtypeStruct((NP, 16), f32),
        ],
    )(x, W1, a_s1, a_d1)

    acc1a, acc1b = _edge_pass1(t1, aux1, edges)

    w2acc = D_OUT + 16
    t2, aux2 = pl.pallas_call(
        _mid_body,
        grid=(g1,),
        in_specs=[
            pl.BlockSpec((blk, w1acc), lambda i: (i, 0)),
            pl.BlockSpec((blk, w1acc), lambda i: (i, 0)),
            pl.BlockSpec((16, D1), lambda i: (0, 0)),
            pl.BlockSpec((1, D1), lambda i: (0, 0)),
            pl.BlockSpec((D1, D_OUT), lambda i: (0, 0)),
            pl.BlockSpec((D_OUT, 16), lambda i: (0, 0)),
        ],
        out_specs=[
            pl.BlockSpec((blk, w2acc), lambda i: (i, 0)),
            pl.BlockSpec((blk, 16), lambda i: (i, 0)),
        ],
        out_shape=[
            jax.ShapeDtypeStruct((NP, w2acc), f32),
            jax.ShapeDtypeStruct((NP, 16), f32),
        ],
    )(acc1a, acc1b, r1, b1r, W2, a2)

    acc2a, acc2b = _edge_pass2(t2, aux2, edges)

    blk_c = 1000
    g2 = N // blk_c
    h, logp = pl.pallas_call(
        _final_body,
        grid=(g2,),
        in_specs=[
            pl.BlockSpec((blk_c, w2acc), lambda i: (i, 0)),
            pl.BlockSpec((blk_c, w2acc), lambda i: (i, 0)),
            pl.BlockSpec((16, D_OUT), lambda i: (0, 0)),
            pl.BlockSpec((1, D_OUT), lambda i: (0, 0)),
        ],
        out_specs=[
            pl.BlockSpec((blk_c, D_OUT), lambda i: (i, 0)),
            pl.BlockSpec((blk_c, D_OUT), lambda i: (i, 0)),
        ],
        out_shape=[
            jax.ShapeDtypeStruct((N, D_OUT), f32),
            jax.ShapeDtypeStruct((N, D_OUT), f32),
        ],
    )(acc2a, acc2b, r2, b2r)

    return (h, logp)
